# baseline (speedup 1.0000x reference)
import functools
import math

import jax
import jax.numpy as jnp
from jax import lax
from jax.experimental import pallas as pl
from jax.experimental.pallas import tpu as pltpu
from jax.experimental.pallas import tpu_sc as plsc

VOCAB = 1000000
D_MODEL = 64
B = 4096
L = 200
N_TOK = B * L
SCALE = math.sqrt(D_MODEL)

NC = 2
NS = 16
NW = NC * NS
PER_W = N_TOK // NW
CHUNK = 128
NCHUNK = PER_W // CHUNK
KB = CHUNK // 128
DPAD = 128

_mesh = plsc.VectorSubcoreMesh(core_axis_name="c", subcore_axis_name="s")


TC_CB = 512
TC_GRID = -(-VOCAB // TC_CB)


def _ttable_body(wt_ref, out_ref):
    out_ref[:, pl.ds(0, D_MODEL)] = wt_ref[...].T * SCALE


_ttable = pl.pallas_call(
    _ttable_body,
    grid=(TC_GRID,),
    in_specs=[pl.BlockSpec((D_MODEL, TC_CB), lambda i: (0, i))],
    out_specs=pl.BlockSpec((TC_CB, DPAD), lambda i: (i, 0)),
    out_shape=jax.ShapeDtypeStruct((VOCAB, DPAD), jnp.float32),
)


@functools.partial(
    pl.kernel,
    mesh=_mesh,
    compiler_params=pltpu.CompilerParams(
        use_tc_tiling_on_sc=False, needs_layout_passes=False
    ),
    out_type=jax.ShapeDtypeStruct((L, 8, B // 128, 8, 128), jnp.float32),
    scratch_types=[
        pltpu.VMEM((PER_W,), jnp.int32),
        pltpu.VMEM((2, CHUNK, DPAD), jnp.float32),
        pltpu.VMEM((2, 8, KB, 8, 128), jnp.float32),
        pltpu.VMEM((CHUNK * 65,), jnp.float32),
    ]
    + [pltpu.SemaphoreType.DMA] * 4,
)
def _embed(idx_hbm, table_hbm, out_hbm, idx_v, bufs, tbufs, skew, *sems):
    gsem = sems[:2]
    osem = sems[2:]
    wid = lax.axis_index("s") * NC + lax.axis_index("c")
    base = wid * PER_W
    pltpu.sync_copy(idx_hbm.at[pl.ds(base, PER_W)], idx_v)

    def fire_gather(g, b):
        off = pl.multiple_of(g * CHUNK, 8)
        pltpu.async_copy(
            table_hbm.at[idx_v.at[pl.ds(off, CHUNK)]], bufs.at[b], gsem[b]
        )

    def wait_gather(g, b):
        off = pl.multiple_of(g * CHUNK, 8)
        pltpu.make_async_copy(
            table_hbm.at[idx_v.at[pl.ds(off, CHUNK)]], bufs.at[b], gsem[b]
        ).wait()

    def out_l_kb0(g):
        off = base + g * CHUNK
        l = off // B
        return l, (off % B) // 128

    def fire_out(g, b):
        l, kb0 = out_l_kb0(g)
        for kd in range(8):
            pltpu.async_copy(
                tbufs.at[b, kd], out_hbm.at[l, kd, pl.ds(kb0, KB)], osem[b]
            )

    def wait_out(g, b):
        l, kb0 = out_l_kb0(g)
        for kd in range(8):
            pltpu.make_async_copy(
                tbufs.at[b, kd], out_hbm.at[l, kd, pl.ds(kb0, KB)], osem[b]
            ).wait()

    iota65 = jax.lax.iota(jnp.int32, 16) * 65

    def transpose_scale(b):
        @plsc.parallel_loop(0, CHUNK, unroll=4)
        def _(r):
            for j in range(D_MODEL // 16):
                skew[pl.ds(r * 65 + j * 16, 16)] = bufs[b, r, pl.ds(j * 16, 16)]

        for kd in range(8):
            for kb in range(KB):
                @plsc.parallel_loop(0, 8, unroll=4)
                def _(q, _kd=kd, _kb=kb):
                    base16 = iota65 + (65 * (_kb * 128) + _kd * 8) + q
                    for j in range(8):
                        v = plsc.load_gather(skew, [base16 + 65 * (j * 16)])
                        tbufs[b, _kd, _kb, q, pl.ds(j * 16, 16)] = v

    fire_gather(0, 0)
    fire_gather(1, 1)

    def step(gi, carry):
        for b in range(2):
            g = gi * 2 + b
            wait_gather(g, b)

            @pl.when(gi >= 1)
            def _():
                wait_out(g - 2, b)

            transpose_scale(b)
            fire_out(g, b)

            @pl.when(gi < (NCHUNK // 2) - 1)
            def _():
                fire_gather(g + 2, b)
        return carry

    lax.fori_loop(0, NCHUNK // 2, step, 0)

    wait_out(NCHUNK - 2, 0)
    wait_out(NCHUNK - 1, 1)


def kernel(x, W):
    idx = x.T.reshape(-1).astype(jnp.int32)
    table = _ttable(W.T)
    out5 = _embed(idx, table)
    return out5.transpose(2, 4, 0, 1, 3).reshape(B, L, D_MODEL)


# speedup vs baseline: 1.2824x; 1.2824x over previous
import functools
import math

import jax
import jax.numpy as jnp
from jax import lax
from jax.experimental import pallas as pl
from jax.experimental.pallas import tpu as pltpu
from jax.experimental.pallas import tpu_sc as plsc

VOCAB = 1000000
D_MODEL = 64
B = 4096
L = 200
N_TOK = B * L
SCALE = math.sqrt(D_MODEL)

NC = 2
NS = 16
NW = NC * NS
PER_W = N_TOK // NW
CHUNK = 128
NCHUNK = PER_W // CHUNK
KB = CHUNK // 128
DPAD = 128

_mesh = plsc.VectorSubcoreMesh(core_axis_name="c", subcore_axis_name="s")


TC_CB = 1024
TC_GRID = -(-VOCAB // TC_CB)


def _ttable_body(wt_ref, out_ref):
    row = jax.lax.broadcasted_iota(jnp.int32, (D_MODEL, D_MODEL), 0)
    col = jax.lax.broadcasted_iota(jnp.int32, (D_MODEL, D_MODEL), 1)
    eye8 = jnp.where(row == col, SCALE, 0.0).astype(jnp.float32)
    out_ref[:, pl.ds(0, D_MODEL)] = jax.lax.dot_general(
        wt_ref[...],
        eye8,
        (((0,), (0,)), ((), ())),
        preferred_element_type=jnp.float32,
        precision=jax.lax.Precision.HIGHEST,
    )


_ttable = pl.pallas_call(
    _ttable_body,
    grid=(TC_GRID,),
    in_specs=[pl.BlockSpec((D_MODEL, TC_CB), lambda i: (0, i))],
    out_specs=pl.BlockSpec((TC_CB, DPAD), lambda i: (i, 0)),
    out_shape=jax.ShapeDtypeStruct((VOCAB, DPAD), jnp.float32),
)


@functools.partial(
    pl.kernel,
    mesh=_mesh,
    compiler_params=pltpu.CompilerParams(
        use_tc_tiling_on_sc=False, needs_layout_passes=False
    ),
    out_type=jax.ShapeDtypeStruct((L, 8, B // 128, 8, 128), jnp.float32),
    scratch_types=[
        pltpu.VMEM((PER_W,), jnp.int32),
        pltpu.VMEM((2, CHUNK, DPAD), jnp.float32),
        pltpu.VMEM((2, 8, KB, 8, 128), jnp.float32),
        pltpu.VMEM((CHUNK * 65,), jnp.float32),
    ]
    + [pltpu.SemaphoreType.DMA] * 4,
)
def _embed(idx_hbm, table_hbm, out_hbm, idx_v, bufs, tbufs, skew, *sems):
    gsem = sems[:2]
    osem = sems[2:]
    wid = lax.axis_index("s") * NC + lax.axis_index("c")
    base = wid * PER_W
    pltpu.sync_copy(idx_hbm.at[pl.ds(base, PER_W)], idx_v)

    def fire_gather(g, b):
        off = pl.multiple_of(g * CHUNK, 8)
        pltpu.async_copy(
            table_hbm.at[idx_v.at[pl.ds(off, CHUNK)]], bufs.at[b], gsem[b]
        )

    def wait_gather(g, b):
        off = pl.multiple_of(g * CHUNK, 8)
        pltpu.make_async_copy(
            table_hbm.at[idx_v.at[pl.ds(off, CHUNK)]], bufs.at[b], gsem[b]
        ).wait()

    def out_l_kb0(g):
        off = base + g * CHUNK
        l = off // B
        return l, (off % B) // 128

    def fire_out(g, b):
        l, kb0 = out_l_kb0(g)
        for kd in range(8):
            pltpu.async_copy(
                tbufs.at[b, kd], out_hbm.at[l, kd, pl.ds(kb0, KB)], osem[b]
            )

    def wait_out(g, b):
        l, kb0 = out_l_kb0(g)
        for kd in range(8):
            pltpu.make_async_copy(
                tbufs.at[b, kd], out_hbm.at[l, kd, pl.ds(kb0, KB)], osem[b]
            ).wait()

    iota65 = jax.lax.iota(jnp.int32, 16) * 65

    def transpose_scale(b):
        @plsc.parallel_loop(0, CHUNK, unroll=4)
        def _(r):
            for j in range(D_MODEL // 16):
                skew[pl.ds(r * 65 + j * 16, 16)] = bufs[b, r, pl.ds(j * 16, 16)]

        for kd in range(8):
            for kb in range(KB):
                @plsc.parallel_loop(0, 8, unroll=4)
                def _(q, _kd=kd, _kb=kb):
                    base16 = iota65 + (65 * (_kb * 128) + _kd * 8) + q
                    for j in range(8):
                        v = plsc.load_gather(skew, [base16 + 65 * (j * 16)])
                        tbufs[b, _kd, _kb, q, pl.ds(j * 16, 16)] = v

    fire_gather(0, 0)
    fire_gather(1, 1)

    def step(gi, carry):
        for b in range(2):
            g = gi * 2 + b
            wait_gather(g, b)

            @pl.when(gi >= 1)
            def _():
                wait_out(g - 2, b)

            transpose_scale(b)
            fire_out(g, b)

            @pl.when(gi < (NCHUNK // 2) - 1)
            def _():
                fire_gather(g + 2, b)
        return carry

    lax.fori_loop(0, NCHUNK // 2, step, 0)

    wait_out(NCHUNK - 2, 0)
    wait_out(NCHUNK - 1, 1)


def kernel(x, W):
    idx = x.T.reshape(-1).astype(jnp.int32)
    table = _ttable(W.T)
    out5 = _embed(idx, table)
    return out5.transpose(2, 4, 0, 1, 3).reshape(B, L, D_MODEL)


# speedup vs baseline: 1.4353x; 1.1192x over previous
import functools
import math

import jax
import jax.numpy as jnp
from jax import lax
from jax.experimental import pallas as pl
from jax.experimental.pallas import tpu as pltpu
from jax.experimental.pallas import tpu_sc as plsc

VOCAB = 1000000
D_MODEL = 64
B = 4096
L = 200
N_TOK = B * L
SCALE = math.sqrt(D_MODEL)

NC = 2
NS = 16
NW = NC * NS
PER_W = N_TOK // NW
CHUNK = 128
NCHUNK = PER_W // CHUNK
KB = CHUNK // 128
DPAD = 128

_mesh = plsc.VectorSubcoreMesh(core_axis_name="c", subcore_axis_name="s")


TC_CB = 1024
TC_GRID = -(-VOCAB // TC_CB)


def _ttable_body(wt_ref, out_ref):
    row = jax.lax.broadcasted_iota(jnp.int32, (D_MODEL, D_MODEL), 0)
    col = jax.lax.broadcasted_iota(jnp.int32, (D_MODEL, D_MODEL), 1)
    eye8 = jnp.where(row == col, SCALE, 0.0).astype(jnp.float32)
    out_ref[:, pl.ds(0, D_MODEL)] = jax.lax.dot_general(
        wt_ref[...],
        eye8,
        (((0,), (0,)), ((), ())),
        preferred_element_type=jnp.float32,
        precision=jax.lax.Precision.DEFAULT,
    )


_ttable = pl.pallas_call(
    _ttable_body,
    grid=(TC_GRID,),
    in_specs=[pl.BlockSpec((D_MODEL, TC_CB), lambda i: (0, i))],
    out_specs=pl.BlockSpec((TC_CB, DPAD), lambda i: (i, 0)),
    out_shape=jax.ShapeDtypeStruct((VOCAB, DPAD), jnp.float32),
)


@functools.partial(
    pl.kernel,
    mesh=_mesh,
    compiler_params=pltpu.CompilerParams(
        use_tc_tiling_on_sc=False, needs_layout_passes=False
    ),
    out_type=jax.ShapeDtypeStruct((L, 8, B // 128, 8, 128), jnp.float32),
    scratch_types=[
        pltpu.VMEM((PER_W,), jnp.int32),
        pltpu.VMEM((2, CHUNK, DPAD), jnp.float32),
        pltpu.VMEM((2, 8, KB, 8, 128), jnp.float32),
        pltpu.VMEM((CHUNK * 65,), jnp.float32),
    ]
    + [pltpu.SemaphoreType.DMA] * 4,
)
def _embed(idx_hbm, table_hbm, out_hbm, idx_v, bufs, tbufs, skew, *sems):
    gsem = sems[:2]
    osem = sems[2:]
    wid = lax.axis_index("s") * NC + lax.axis_index("c")
    base = wid * PER_W
    pltpu.sync_copy(idx_hbm.at[pl.ds(base, PER_W)], idx_v)

    def fire_gather(g, b):
        off = pl.multiple_of(g * CHUNK, 8)
        pltpu.async_copy(
            table_hbm.at[idx_v.at[pl.ds(off, CHUNK)]], bufs.at[b], gsem[b]
        )

    def wait_gather(g, b):
        off = pl.multiple_of(g * CHUNK, 8)
        pltpu.make_async_copy(
            table_hbm.at[idx_v.at[pl.ds(off, CHUNK)]], bufs.at[b], gsem[b]
        ).wait()

    def out_l_kb0(g):
        off = base + g * CHUNK
        l = off // B
        return l, (off % B) // 128

    def fire_out(g, b):
        l, kb0 = out_l_kb0(g)
        for kd in range(8):
            pltpu.async_copy(
                tbufs.at[b, kd], out_hbm.at[l, kd, pl.ds(kb0, KB)], osem[b]
            )

    def wait_out(g, b):
        l, kb0 = out_l_kb0(g)
        for kd in range(8):
            pltpu.make_async_copy(
                tbufs.at[b, kd], out_hbm.at[l, kd, pl.ds(kb0, KB)], osem[b]
            ).wait()

    iota65 = jax.lax.iota(jnp.int32, 16) * 65

    def transpose_scale(b):
        @plsc.parallel_loop(0, CHUNK, unroll=4)
        def _(r):
            for j in range(D_MODEL // 16):
                skew[pl.ds(r * 65 + j * 16, 16)] = bufs[b, r, pl.ds(j * 16, 16)]

        for kd in range(8):
            for kb in range(KB):
                @plsc.parallel_loop(0, 8, unroll=4)
                def _(q, _kd=kd, _kb=kb):
                    base16 = iota65 + (65 * (_kb * 128) + _kd * 8) + q
                    for j in range(8):
                        v = plsc.load_gather(skew, [base16 + 65 * (j * 16)])
                        tbufs[b, _kd, _kb, q, pl.ds(j * 16, 16)] = v

    fire_gather(0, 0)
    fire_gather(1, 1)

    def step(gi, carry):
        for b in range(2):
            g = gi * 2 + b
            wait_gather(g, b)

            @pl.when(gi >= 1)
            def _():
                wait_out(g - 2, b)

            transpose_scale(b)
            fire_out(g, b)

            @pl.when(gi < (NCHUNK // 2) - 1)
            def _():
                fire_gather(g + 2, b)
        return carry

    lax.fori_loop(0, NCHUNK // 2, step, 0)

    wait_out(NCHUNK - 2, 0)
    wait_out(NCHUNK - 1, 1)


def kernel(x, W):
    idx = x.T.reshape(-1).astype(jnp.int32)
    table = _ttable(W.T)
    out5 = _embed(idx, table)
    return out5.transpose(2, 4, 0, 1, 3).reshape(B, L, D_MODEL)


# speedup vs baseline: 1.6566x; 1.1542x over previous
import functools
import math

import jax
import jax.numpy as jnp
from jax import lax
from jax.experimental import pallas as pl
from jax.experimental.pallas import tpu as pltpu
from jax.experimental.pallas import tpu_sc as plsc

VOCAB = 1000000
D_MODEL = 64
B = 4096
L = 200
N_TOK = B * L
SCALE = math.sqrt(D_MODEL)

NC = 2
NS = 16
NW = NC * NS
PER_W = N_TOK // NW
CHUNK = 256
NCHUNK = PER_W // CHUNK
KB = CHUNK // 128

_mesh = plsc.VectorSubcoreMesh(core_axis_name="c", subcore_axis_name="s")

@functools.partial(
    pl.kernel,
    mesh=_mesh,
    compiler_params=pltpu.CompilerParams(
        use_tc_tiling_on_sc=False, needs_layout_passes=False
    ),
    out_type=jax.ShapeDtypeStruct((L, 8, B // 128, 8, 128), jnp.float32),
    scratch_types=[
        pltpu.VMEM((PER_W,), jnp.int32),
        pltpu.VMEM((2, CHUNK, D_MODEL), jnp.float32),
        pltpu.VMEM((2, 8, KB, 8, 128), jnp.float32),
        pltpu.VMEM((CHUNK * 65,), jnp.float32),
    ]
    + [pltpu.SemaphoreType.DMA] * 4,
)
def _embed(idx_hbm, table_hbm, out_hbm, idx_v, bufs, tbufs, skew, *sems):
    gsem = sems[:2]
    osem = sems[2:]
    wid = lax.axis_index("s") * NC + lax.axis_index("c")
    base = wid * PER_W
    pltpu.sync_copy(idx_hbm.at[pl.ds(base, PER_W)], idx_v)

    def fire_gather(g, b):
        off = pl.multiple_of(g * CHUNK, 8)
        pltpu.async_copy(
            table_hbm.at[idx_v.at[pl.ds(off, CHUNK)]], bufs.at[b], gsem[b]
        )

    def wait_gather(g, b):
        off = pl.multiple_of(g * CHUNK, 8)
        pltpu.make_async_copy(
            table_hbm.at[idx_v.at[pl.ds(off, CHUNK)]], bufs.at[b], gsem[b]
        ).wait()

    def out_l_kb0(g):
        off = base + g * CHUNK
        l = off // B
        return l, (off % B) // 128

    def fire_out(g, b):
        l, kb0 = out_l_kb0(g)
        for kd in range(8):
            pltpu.async_copy(
                tbufs.at[b, kd], out_hbm.at[l, kd, pl.ds(kb0, KB)], osem[b]
            )

    def wait_out(g, b):
        l, kb0 = out_l_kb0(g)
        for kd in range(8):
            pltpu.make_async_copy(
                tbufs.at[b, kd], out_hbm.at[l, kd, pl.ds(kb0, KB)], osem[b]
            ).wait()

    iota65 = jax.lax.iota(jnp.int32, 16) * 65

    def transpose_scale(b):
        @plsc.parallel_loop(0, CHUNK, unroll=4)
        def _(r):
            for j in range(D_MODEL // 16):
                skew[pl.ds(r * 65 + j * 16, 16)] = bufs[b, r, pl.ds(j * 16, 16)]

        for kd in range(8):
            for kb in range(KB):
                @plsc.parallel_loop(0, 8, unroll=4)
                def _(q, _kd=kd, _kb=kb):
                    base16 = iota65 + (65 * (_kb * 128) + _kd * 8) + q
                    for j in range(8):
                        v = plsc.load_gather(skew, [base16 + 65 * (j * 16)])
                        tbufs[b, _kd, _kb, q, pl.ds(j * 16, 16)] = v * SCALE

    fire_gather(0, 0)
    fire_gather(1, 1)

    def step(gi, carry):
        for b in range(2):
            g = gi * 2 + b
            wait_gather(g, b)

            @pl.when(gi >= 1)
            def _():
                wait_out(g - 2, b)

            transpose_scale(b)
            fire_out(g, b)

            @pl.when(gi < (NCHUNK // 2) - 1)
            def _():
                fire_gather(g + 2, b)
        return carry

    lax.fori_loop(0, NCHUNK // 2, step, 0)

    wait_out(NCHUNK - 2, 0)
    wait_out(NCHUNK - 1, 1)


def kernel(x, W):
    idx = x.T.reshape(-1).astype(jnp.int32)
    out5 = _embed(idx, W)
    return out5.transpose(2, 4, 0, 1, 3).reshape(B, L, D_MODEL)
